# baseline (speedup 1.0000x reference)
import functools

import jax
import jax.numpy as jnp
from jax import lax
from jax.experimental import pallas as pl
from jax.experimental.pallas import tpu as pltpu
from jax.experimental.pallas import tpu_sc as plsc

D = 64
NC = 2
NS = 16
NW = NC * NS
CH = 512


@functools.partial(jax.jit, static_argnames=("n_flat",))
def _gather_flat(table, idx_flat, n_flat):
    bpw = n_flat // NW
    nchunk = bpw // CH
    npair = (nchunk - 2) // 2

    @functools.partial(
        pl.kernel,
        mesh=plsc.VectorSubcoreMesh(core_axis_name="c", subcore_axis_name="s"),
        out_type=jax.ShapeDtypeStruct((n_flat, D), jnp.float32),
        scratch_types=[
            pltpu.VMEM((2, CH), jnp.int32),
            pltpu.VMEM((2, CH, D), jnp.float32),
            pltpu.SemaphoreType.DMA,
            pltpu.SemaphoreType.DMA,
            pltpu.SemaphoreType.DMA,
            pltpu.SemaphoreType.DMA,
            pltpu.SemaphoreType.DMA,
            pltpu.SemaphoreType.DMA,
        ],
        compiler_params=pltpu.CompilerParams(use_tc_tiling_on_sc=False),
    )
    def body(table_hbm, idx_hbm, out_hbm, idx_v, rows_v,
             si0, si1, sg0, sg1, so0, so1):
        sem_i, sem_g, sem_o = (si0, si1), (sg0, sg1), (so0, so1)
        wid = lax.axis_index("s") * NC + lax.axis_index("c")
        base = wid * bpw

        def idx_cp(c, b):
            return pltpu.make_async_copy(
                idx_hbm.at[pl.ds(base + c * CH, CH)], idx_v.at[b], sem_i[b])

        def gat_cp(b):
            return pltpu.make_async_copy(
                table_hbm.at[idx_v.at[b]], rows_v.at[b], sem_g[b])

        def out_cp(c, b):
            return pltpu.make_async_copy(
                rows_v.at[b], out_hbm.at[pl.ds(base + c * CH, CH)], sem_o[b])

        idx_cp(0, 0).start()
        idx_cp(1, 1).start()
        idx_cp(0, 0).wait()
        gat_cp(0).start()
        idx_cp(1, 1).wait()
        gat_cp(1).start()

        def pair(g2, carry):
            for b in (0, 1):
                c = 2 * g2 + b
                gat_cp(b).wait()
                out_cp(c, b).start()
                idx_cp(c + 2, b).start()
                out_cp(c, b).wait()
                idx_cp(c + 2, b).wait()
                gat_cp(b).start()
            return carry

        lax.fori_loop(0, npair, pair, 0)

        for b, c in ((0, nchunk - 2), (1, nchunk - 1)):
            gat_cp(b).wait()
            out_cp(c, b).start()
        for b, c in ((0, nchunk - 2), (1, nchunk - 1)):
            out_cp(c, b).wait()

    return body(table, idx_flat)


def kernel(data, table):
    n_flat = data.size
    out = _gather_flat(table, data.reshape(-1), n_flat)
    return out.reshape(data.shape + (table.shape[1],))


# speedup vs baseline: 1.1686x; 1.1686x over previous
import functools

import jax
import jax.numpy as jnp
from jax import lax
from jax.experimental import pallas as pl
from jax.experimental.pallas import tpu as pltpu
from jax.experimental.pallas import tpu_sc as plsc
from jax.experimental.layout import Layout, with_layout_constraint

D = 64
NC = 2
NS = 16
NW = NC * NS
CH = 512


@functools.partial(jax.jit, static_argnames=("n_flat",))
def _gather_flat(table, idx_flat, n_flat):
    bpw = n_flat // NW
    nchunk = bpw // CH
    npair = (nchunk - 2) // 2

    @functools.partial(
        pl.kernel,
        mesh=plsc.VectorSubcoreMesh(core_axis_name="c", subcore_axis_name="s"),
        out_type=jax.ShapeDtypeStruct((n_flat, D), jnp.float32),
        scratch_types=[
            pltpu.VMEM((2, CH), jnp.int32),
            pltpu.VMEM((2, CH, D), jnp.float32),
            pltpu.SemaphoreType.DMA,
            pltpu.SemaphoreType.DMA,
            pltpu.SemaphoreType.DMA,
            pltpu.SemaphoreType.DMA,
            pltpu.SemaphoreType.DMA,
            pltpu.SemaphoreType.DMA,
        ],
        compiler_params=pltpu.CompilerParams(use_tc_tiling_on_sc=False),
    )
    def body(table_hbm, idx_hbm, out_hbm, idx_v, rows_v,
             si0, si1, sg0, sg1, so0, so1):
        sem_i, sem_g, sem_o = (si0, si1), (sg0, sg1), (so0, so1)
        wid = lax.axis_index("s") * NC + lax.axis_index("c")
        base = wid * bpw

        def idx_cp(c, b):
            return pltpu.make_async_copy(
                idx_hbm.at[pl.ds(base + c * CH, CH)], idx_v.at[b], sem_i[b])

        def gat_cp(b):
            return pltpu.make_async_copy(
                table_hbm.at[idx_v.at[b]], rows_v.at[b], sem_g[b])

        def out_cp(c, b):
            return pltpu.make_async_copy(
                rows_v.at[b], out_hbm.at[pl.ds(base + c * CH, CH)], sem_o[b])

        idx_cp(0, 0).start()
        idx_cp(1, 1).start()
        idx_cp(0, 0).wait()
        gat_cp(0).start()
        idx_cp(1, 1).wait()
        gat_cp(1).start()

        def pair(g2, carry):
            for b in (0, 1):
                c = 2 * g2 + b
                gat_cp(b).wait()
                out_cp(c, b).start()
                idx_cp(c + 2, b).start()
                out_cp(c, b).wait()
                idx_cp(c + 2, b).wait()
                gat_cp(b).start()
            return carry

        lax.fori_loop(0, npair, pair, 0)

        for b, c in ((0, nchunk - 2), (1, nchunk - 1)):
            gat_cp(b).wait()
            out_cp(c, b).start()
        for b, c in ((0, nchunk - 2), (1, nchunk - 1)):
            out_cp(c, b).wait()

    return body(table, idx_flat)


def kernel(data, table):
    n_flat = data.size
    out = _gather_flat(table, data.reshape(-1), n_flat)
    out = out.reshape(data.shape + (table.shape[1],))
    return with_layout_constraint(
        out, Layout(major_to_minor=tuple(range(out.ndim))))
